# Initial kernel scaffold; baseline (speedup 1.0000x reference)
import jax
import jax.numpy as jnp
from jax.experimental import pallas as pl


def kernel(x, edge_index, edge_attr, W1, b1, W2, b2, W3, b3, W4, b4):
    raise NotImplementedError("write your pallas kernel here")


# speedup vs baseline: 1.4600x; 1.4600x over previous
import functools

import jax
import jax.numpy as jnp
from jax import lax
from jax.experimental import pallas as pl
from jax.experimental.pallas import tpu as pltpu
from jax.experimental.pallas import tpu_sc as plsc

N_NODES = 10000
N_EDGES = 320000
D_FEAT = 128
D_EDGE = 16
IN_SIZE = D_FEAT + D_EDGE
HID = IN_SIZE * 2
HALF = HID // 2
NVEC = HALF // 16

NSUB = 16
EDGES_PER_TILE = N_EDGES // NSUB
EB = 80
NBLK = EDGES_PER_TILE // EB
N_PAD = 10240
ROWS_PER_TILE = N_PAD // NSUB

NODE_BLK = 1000
EDGE_BLK = 8000


def _xw_body(x_ref, w_ref, b_ref, oa_ref, ob_ref):
    r = jnp.dot(x_ref[...], w_ref[...], preferred_element_type=jnp.float32)
    r = r + b_ref[...]
    oa_ref[...] = r[:, :HALF]
    ob_ref[...] = r[:, HALF:]


def _ew_body(ea_ref, w_ref, oa_ref, ob_ref):
    r = jnp.dot(ea_ref[...], w_ref[...], preferred_element_type=jnp.float32)
    oa_ref[...] = r[:, :HALF]
    ob_ref[...] = r[:, HALF:]


_SC_MESH = plsc.VectorSubcoreMesh(core_axis_name="c", subcore_axis_name="s")


@functools.partial(
    pl.kernel,
    out_type=[
        jax.ShapeDtypeStruct((N_PAD, HALF), jnp.float32),
        jax.ShapeDtypeStruct((N_PAD, HALF), jnp.float32),
        jax.ShapeDtypeStruct((N_PAD, 16), jnp.float32),
    ],
    mesh=_SC_MESH,
    compiler_params=pltpu.CompilerParams(use_tc_tiling_on_sc=False),
    scratch_types=[
        pltpu.VMEM((EB,), jnp.int32),
        pltpu.VMEM((EB,), jnp.int32),
        pltpu.VMEM((EB, HALF), jnp.float32),
        pltpu.VMEM((EB, HALF), jnp.float32),
        pltpu.VMEM((EB, 16), jnp.float32),
        pltpu.VMEM_SHARED((N_PAD, HALF), jnp.float32),
        pltpu.VMEM_SHARED((N_PAD, 16), jnp.float32),
        pltpu.SemaphoreType.DMA,
        pltpu.SemaphoreType.DMA,
    ],
)
def _sc_aggregate(xwa_h, xwb_h, ewa_h, ewb_h, row_h, col_h, zrow_h, zcnt_h,
                  ones_h, ra_o, rb_o, cnt_o,
                  row_v, col_v, gbuf, ebuf, ones_v, acc, cacc, sem1, sem2):
    c = lax.axis_index("c")
    s = lax.axis_index("s")
    r0 = pl.multiple_of(s * ROWS_PER_TILE, 8)

    pltpu.sync_copy(zrow_h, acc.at[pl.ds(r0, ROWS_PER_TILE)])
    pltpu.sync_copy(ones_h, ones_v)

    @pl.when(c == 0)
    def _():
        pltpu.sync_copy(zcnt_h, cacc.at[pl.ds(r0, ROWS_PER_TILE)])

    plsc.subcore_barrier()

    def run_edges(xw_h, ew_h, do_cnt):
        def body(blk, carry):
            e0 = pl.multiple_of(s * EDGES_PER_TILE + blk * EB, 8)
            pltpu.sync_copy(row_h.at[pl.ds(e0, EB)], row_v)
            pltpu.sync_copy(col_h.at[pl.ds(e0, EB)], col_v)
            cp1 = pltpu.async_copy(xw_h.at[row_v], gbuf, sem1)
            cp2 = pltpu.async_copy(ew_h.at[pl.ds(e0, EB)], ebuf, sem2)
            cp1.wait()
            cp2.wait()

            def row_body(j, cr):
                for k in range(NVEC):
                    sl = pl.ds(k * 16, 16)
                    gbuf[j, sl] = jnp.maximum(gbuf[j, sl] + ebuf[j, sl], 0.0)
                return cr

            lax.fori_loop(0, EB, row_body, 0)
            pltpu.sync_copy(gbuf, acc.at[col_v], add=True)
            if do_cnt:
                pltpu.sync_copy(ones_v, cacc.at[col_v], add=True)
            return carry

        lax.fori_loop(0, NBLK, body, 0)

    @pl.when(c == 0)
    def _():
        run_edges(xwa_h, ewa_h, True)

    @pl.when(c == 1)
    def _():
        run_edges(xwb_h, ewb_h, False)

    plsc.subcore_barrier()

    @pl.when(c == 0)
    def _():
        pltpu.sync_copy(acc.at[pl.ds(r0, ROWS_PER_TILE)],
                        ra_o.at[pl.ds(r0, ROWS_PER_TILE)])
        pltpu.sync_copy(cacc.at[pl.ds(r0, ROWS_PER_TILE)],
                        cnt_o.at[pl.ds(r0, ROWS_PER_TILE)])

    @pl.when(c == 1)
    def _():
        pltpu.sync_copy(acc.at[pl.ds(r0, ROWS_PER_TILE)],
                        rb_o.at[pl.ds(r0, ROWS_PER_TILE)])


def _node_body(x_ref, ra_ref, rb_ref, cnt_ref, w2a_ref, w2b_ref, b2_ref,
               w3x_ref, w3m_ref, b3_ref, w4_ref, b4_ref, o_ref):
    cnt = cnt_ref[:, 0:1]
    inv = 1.0 / jnp.maximum(cnt, 1.0)
    gate = jnp.minimum(cnt, 1.0)
    m = jnp.dot(ra_ref[...] * inv, w2a_ref[...],
                preferred_element_type=jnp.float32)
    m = m + jnp.dot(rb_ref[...] * inv, w2b_ref[...],
                    preferred_element_type=jnp.float32)
    m = m + gate * b2_ref[...]
    h = jnp.dot(x_ref[...], w3x_ref[...], preferred_element_type=jnp.float32)
    h = h + jnp.dot(m, w3m_ref[...], preferred_element_type=jnp.float32)
    h = jnp.maximum(h + b3_ref[...], 0.0)
    o_ref[...] = jnp.dot(h, w4_ref[...],
                         preferred_element_type=jnp.float32) + b4_ref[...]


@jax.jit
def kernel(x, edge_index, edge_attr, W1, b1, W2, b2, W3, b3, W4, b4):
    row = edge_index[0].astype(jnp.int32)
    col = edge_index[1].astype(jnp.int32)

    xwa, xwb = pl.pallas_call(
        _xw_body,
        grid=(N_NODES // NODE_BLK,),
        in_specs=[
            pl.BlockSpec((NODE_BLK, D_FEAT), lambda i: (i, 0)),
            pl.BlockSpec((D_FEAT, HID), lambda i: (0, 0)),
            pl.BlockSpec((1, HID), lambda i: (0, 0)),
        ],
        out_specs=[
            pl.BlockSpec((NODE_BLK, HALF), lambda i: (i, 0)),
            pl.BlockSpec((NODE_BLK, HALF), lambda i: (i, 0)),
        ],
        out_shape=[
            jax.ShapeDtypeStruct((N_NODES, HALF), jnp.float32),
            jax.ShapeDtypeStruct((N_NODES, HALF), jnp.float32),
        ],
    )(x, W1[:D_FEAT], b1.reshape(1, HID))

    ewa, ewb = pl.pallas_call(
        _ew_body,
        grid=(N_EDGES // EDGE_BLK,),
        in_specs=[
            pl.BlockSpec((EDGE_BLK, D_EDGE), lambda i: (i, 0)),
            pl.BlockSpec((D_EDGE, HID), lambda i: (0, 0)),
        ],
        out_specs=[
            pl.BlockSpec((EDGE_BLK, HALF), lambda i: (i, 0)),
            pl.BlockSpec((EDGE_BLK, HALF), lambda i: (i, 0)),
        ],
        out_shape=[
            jax.ShapeDtypeStruct((N_EDGES, HALF), jnp.float32),
            jax.ShapeDtypeStruct((N_EDGES, HALF), jnp.float32),
        ],
    )(edge_attr, W1[D_FEAT:])

    zrow = jnp.zeros((ROWS_PER_TILE, HALF), jnp.float32)
    zcnt = jnp.zeros((ROWS_PER_TILE, 16), jnp.float32)
    ones = jnp.ones((EB, 16), jnp.float32)
    ra, rb, cnt = _sc_aggregate(xwa, xwb, ewa, ewb, row, col, zrow, zcnt, ones)

    out = pl.pallas_call(
        _node_body,
        grid=(N_NODES // NODE_BLK,),
        in_specs=[
            pl.BlockSpec((NODE_BLK, D_FEAT), lambda i: (i, 0)),
            pl.BlockSpec((NODE_BLK, HALF), lambda i: (i, 0)),
            pl.BlockSpec((NODE_BLK, HALF), lambda i: (i, 0)),
            pl.BlockSpec((NODE_BLK, 16), lambda i: (i, 0)),
            pl.BlockSpec((HALF, HID), lambda i: (0, 0)),
            pl.BlockSpec((HALF, HID), lambda i: (0, 0)),
            pl.BlockSpec((1, HID), lambda i: (0, 0)),
            pl.BlockSpec((D_FEAT, IN_SIZE), lambda i: (0, 0)),
            pl.BlockSpec((HID, IN_SIZE), lambda i: (0, 0)),
            pl.BlockSpec((1, IN_SIZE), lambda i: (0, 0)),
            pl.BlockSpec((IN_SIZE, D_FEAT), lambda i: (0, 0)),
            pl.BlockSpec((1, D_FEAT), lambda i: (0, 0)),
        ],
        out_specs=pl.BlockSpec((NODE_BLK, D_FEAT), lambda i: (i, 0)),
        out_shape=jax.ShapeDtypeStruct((N_NODES, D_FEAT), jnp.float32),
    )(x, ra, rb, cnt, W2[:HALF], W2[HALF:], b2.reshape(1, HID),
      W3[:D_FEAT], W3[D_FEAT:], b3.reshape(1, IN_SIZE), W4,
      b4.reshape(1, D_FEAT))
    return out


# speedup vs baseline: 1.8388x; 1.2595x over previous
import functools

import jax
import jax.numpy as jnp
from jax import lax
from jax.experimental import pallas as pl
from jax.experimental.pallas import tpu as pltpu
from jax.experimental.pallas import tpu_sc as plsc

N_NODES = 10000
N_EDGES = 320000
D_FEAT = 128
D_EDGE = 16
IN_SIZE = D_FEAT + D_EDGE
HID = IN_SIZE * 2
CW = 128
CNT_COL = HID - 2 * CW

NSUB = 16
EDGES_PER_CORE = N_EDGES // 2
EDGES_PER_TILE = EDGES_PER_CORE // NSUB
EB = 80
NBLK = EDGES_PER_TILE // EB
N_PAD = 10240
ROWS_PER_TILE = N_PAD // NSUB

HID_PAD = 3 * CW
PASSES = ((0, 8, 8), (CW, 8, 8), (2 * CW, 2, 3))

NODE_BLK = 1000
EDGE_BLK = 8000


def _xw_body(x_ref, w_ref, b_ref, o0_ref, o1_ref, o2_ref):
    r = jnp.dot(x_ref[...], w_ref[...], preferred_element_type=jnp.float32)
    r = r + b_ref[...]
    n = r.shape[0]
    o0_ref[...] = r[:, :CW]
    o1_ref[...] = r[:, CW:2 * CW]
    o2_ref[...] = jnp.concatenate(
        [r[:, 2 * CW:], jnp.ones((n, 1), jnp.float32),
         jnp.zeros((n, CW - CNT_COL - 1), jnp.float32)], axis=1)


def _ew_body(ea_ref, w_ref, o_ref):
    o_ref[...] = jnp.dot(ea_ref[...], w_ref[...],
                         preferred_element_type=jnp.float32)


_SC_MESH = plsc.VectorSubcoreMesh(core_axis_name="c", subcore_axis_name="s")


@functools.partial(
    pl.kernel,
    out_type=[
        jax.ShapeDtypeStruct((2, N_PAD, CW), jnp.float32),
        jax.ShapeDtypeStruct((2, N_PAD, CW), jnp.float32),
        jax.ShapeDtypeStruct((2, N_PAD, CW), jnp.float32),
    ],
    mesh=_SC_MESH,
    scratch_types=[
        pltpu.VMEM((EB,), jnp.int32),
        pltpu.VMEM((EB,), jnp.int32),
        pltpu.VMEM((EB, CW), jnp.float32),
        pltpu.VMEM((EB, CW), jnp.float32),
        pltpu.VMEM_SHARED((N_PAD, CW), jnp.float32),
        pltpu.SemaphoreType.DMA,
        pltpu.SemaphoreType.DMA,
    ],
)
def _sc_aggregate(xw0_h, xw1_h, xw2_h, ew_h, row_h, col_h, zrow_h,
                  p0_o, p1_o, p2_o,
                  row_v, col_v, gbuf, ebuf, acc, sem1, sem2):
    c = lax.axis_index("c")
    s = lax.axis_index("s")
    r0 = pl.multiple_of(s * ROWS_PER_TILE, 8)

    def run_pass(xw_h, ew_off, ew_nv, nv, out_ref):
        pltpu.sync_copy(zrow_h, acc.at[pl.ds(r0, ROWS_PER_TILE)])
        plsc.subcore_barrier()

        def body(blk, carry):
            e0 = pl.multiple_of(
                c * EDGES_PER_CORE + s * EDGES_PER_TILE + blk * EB, 8)
            pltpu.sync_copy(row_h.at[pl.ds(e0, EB)], row_v)
            pltpu.sync_copy(col_h.at[pl.ds(e0, EB)], col_v)
            cp1 = pltpu.async_copy(xw_h.at[row_v], gbuf, sem1)
            cp2 = pltpu.async_copy(
                ew_h.at[pl.ds(e0, EB), pl.ds(ew_off, CW)], ebuf, sem2)
            cp1.wait()
            cp2.wait()

            def row_body(j, cr):
                for k in range(nv):
                    sl = pl.ds(k * 16, 16)
                    g = gbuf[j, sl]
                    if k < ew_nv:
                        g = g + ebuf[j, sl]
                    gbuf[j, sl] = jnp.maximum(g, 0.0)
                return cr

            lax.fori_loop(0, EB, row_body, 0)
            pltpu.sync_copy(gbuf, acc.at[col_v], add=True)
            return carry

        lax.fori_loop(0, NBLK, body, 0)
        plsc.subcore_barrier()
        pltpu.sync_copy(acc.at[pl.ds(r0, ROWS_PER_TILE)],
                        out_ref.at[c, pl.ds(r0, ROWS_PER_TILE)])

    run_pass(xw0_h, PASSES[0][0], PASSES[0][1], PASSES[0][2], p0_o)
    run_pass(xw1_h, PASSES[1][0], PASSES[1][1], PASSES[1][2], p1_o)
    run_pass(xw2_h, PASSES[2][0], PASSES[2][1], PASSES[2][2], p2_o)


def _node_body(x_ref, p0a_ref, p0b_ref, p1a_ref, p1b_ref, p2a_ref, p2b_ref,
               w2a_ref, w2b_ref, w2c_ref, b2_ref,
               w3x_ref, w3m_ref, b3_ref, w4_ref, b4_ref, o_ref):
    r0 = p0a_ref[0] + p0b_ref[0]
    r1 = p1a_ref[0] + p1b_ref[0]
    r2 = p2a_ref[0] + p2b_ref[0]
    cnt = r2[:, CNT_COL:CNT_COL + 1]
    inv = 1.0 / jnp.maximum(cnt, 1.0)
    gate = jnp.minimum(cnt, 1.0)
    m = jnp.dot(r0 * inv, w2a_ref[...], preferred_element_type=jnp.float32)
    m = m + jnp.dot(r1 * inv, w2b_ref[...], preferred_element_type=jnp.float32)
    m = m + jnp.dot(r2[:, :CNT_COL] * inv, w2c_ref[...],
                    preferred_element_type=jnp.float32)
    m = m + gate * b2_ref[...]
    h = jnp.dot(x_ref[...], w3x_ref[...], preferred_element_type=jnp.float32)
    h = h + jnp.dot(m, w3m_ref[...], preferred_element_type=jnp.float32)
    h = jnp.maximum(h + b3_ref[...], 0.0)
    o_ref[...] = jnp.dot(h, w4_ref[...],
                         preferred_element_type=jnp.float32) + b4_ref[...]


@jax.jit
def kernel(x, edge_index, edge_attr, W1, b1, W2, b2, W3, b3, W4, b4):
    row = edge_index[0].astype(jnp.int32)
    col = edge_index[1].astype(jnp.int32)

    xw0, xw1, xw2 = pl.pallas_call(
        _xw_body,
        grid=(N_NODES // NODE_BLK,),
        in_specs=[
            pl.BlockSpec((NODE_BLK, D_FEAT), lambda i: (i, 0)),
            pl.BlockSpec((D_FEAT, HID), lambda i: (0, 0)),
            pl.BlockSpec((1, HID), lambda i: (0, 0)),
        ],
        out_specs=[
            pl.BlockSpec((NODE_BLK, CW), lambda i: (i, 0)),
            pl.BlockSpec((NODE_BLK, CW), lambda i: (i, 0)),
            pl.BlockSpec((NODE_BLK, CW), lambda i: (i, 0)),
        ],
        out_shape=[
            jax.ShapeDtypeStruct((N_NODES, CW), jnp.float32),
            jax.ShapeDtypeStruct((N_NODES, CW), jnp.float32),
            jax.ShapeDtypeStruct((N_NODES, CW), jnp.float32),
        ],
    )(x, W1[:D_FEAT], b1.reshape(1, HID))

    w1e_pad = jnp.concatenate(
        [W1[D_FEAT:], jnp.zeros((D_EDGE, HID_PAD - HID), jnp.float32)], axis=1)
    ew = pl.pallas_call(
        _ew_body,
        grid=(N_EDGES // EDGE_BLK,),
        in_specs=[
            pl.BlockSpec((EDGE_BLK, D_EDGE), lambda i: (i, 0)),
            pl.BlockSpec((D_EDGE, HID_PAD), lambda i: (0, 0)),
        ],
        out_specs=pl.BlockSpec((EDGE_BLK, HID_PAD), lambda i: (i, 0)),
        out_shape=jax.ShapeDtypeStruct((N_EDGES, HID_PAD), jnp.float32),
    )(edge_attr, w1e_pad)

    zrow = jnp.zeros((ROWS_PER_TILE, CW), jnp.float32)
    p0, p1, p2 = _sc_aggregate(xw0, xw1, xw2, ew, row, col, zrow)

    def blk(i):
        return (i, 0)

    out = pl.pallas_call(
        _node_body,
        grid=(N_NODES // NODE_BLK,),
        in_specs=[
            pl.BlockSpec((NODE_BLK, D_FEAT), blk),
            pl.BlockSpec((1, NODE_BLK, CW), lambda i: (0, i, 0)),
            pl.BlockSpec((1, NODE_BLK, CW), lambda i: (1, i, 0)),
            pl.BlockSpec((1, NODE_BLK, CW), lambda i: (0, i, 0)),
            pl.BlockSpec((1, NODE_BLK, CW), lambda i: (1, i, 0)),
            pl.BlockSpec((1, NODE_BLK, CW), lambda i: (0, i, 0)),
            pl.BlockSpec((1, NODE_BLK, CW), lambda i: (1, i, 0)),
            pl.BlockSpec((CW, HID), lambda i: (0, 0)),
            pl.BlockSpec((CW, HID), lambda i: (0, 0)),
            pl.BlockSpec((CNT_COL, HID), lambda i: (0, 0)),
            pl.BlockSpec((1, HID), lambda i: (0, 0)),
            pl.BlockSpec((D_FEAT, IN_SIZE), lambda i: (0, 0)),
            pl.BlockSpec((HID, IN_SIZE), lambda i: (0, 0)),
            pl.BlockSpec((1, IN_SIZE), lambda i: (0, 0)),
            pl.BlockSpec((IN_SIZE, D_FEAT), lambda i: (0, 0)),
            pl.BlockSpec((1, D_FEAT), lambda i: (0, 0)),
        ],
        out_specs=pl.BlockSpec((NODE_BLK, D_FEAT), blk),
        out_shape=jax.ShapeDtypeStruct((N_NODES, D_FEAT), jnp.float32),
    )(x, p0, p0, p1, p1, p2, p2, W2[:CW], W2[CW:2 * CW], W2[2 * CW:],
      b2.reshape(1, HID), W3[:D_FEAT], W3[D_FEAT:], b3.reshape(1, IN_SIZE),
      W4, b4.reshape(1, D_FEAT))
    return out
